# baseline (speedup 1.0000x reference)
import functools

import jax
import jax.numpy as jnp
from jax import lax
from jax.experimental import pallas as pl
from jax.experimental.pallas import tpu as pltpu
from jax.experimental.pallas import tpu_sc as plsc

_N, _C, _H, _W = 128, 3, 224, 224
_NC, _NS = 2, 16
_NW = _NC * _NS
_ROWS = _C * _H
_RPW = _ROWS // _NW
_NBUF = 4
_LA = 2


def _sc_copy(x_hbm, o_hbm, buf, in_sems, out_sems):
    wid = lax.axis_index("s") * _NC + lax.axis_index("c")
    base = wid * _RPW

    def start_in(q):
        r = base + q
        s = q % _NBUF
        return pltpu.async_copy(x_hbm.at[r // _H, r % _H], buf.at[s],
                                in_sems.at[s])

    def start_out(q):
        r = base + q
        s = q % _NBUF
        return pltpu.async_copy(buf.at[s], o_hbm.at[r // _H, r % _H],
                                out_sems.at[s])

    ins, outs = {}, {}
    for q in range(-_LA, _RPW):
        j = q + _LA
        if j < _RPW:
            if j >= _NBUF:
                outs[j - _NBUF].wait()
            ins[j] = start_in(j)
        if q >= 0:
            ins[q].wait()
            outs[q] = start_out(q)
    for q in range(max(_RPW - _NBUF, 0), _RPW):
        outs[q].wait()


def kernel(x):
    y = jnp.transpose(x, (1, 2, 3, 0))
    k = functools.partial(
        pl.kernel,
        mesh=plsc.VectorSubcoreMesh(core_axis_name="c", subcore_axis_name="s"),
        out_type=jax.ShapeDtypeStruct((_C, _H, _W, _N), jnp.float32),
        scratch_types=[
            pltpu.VMEM((_NBUF, _W, _N), jnp.float32),
            pltpu.SemaphoreType.DMA((_NBUF,)),
            pltpu.SemaphoreType.DMA((_NBUF,)),
        ],
    )(_sc_copy)
    out = k(y)
    return jnp.transpose(out, (3, 0, 1, 2))


# speedup vs baseline: 1.5346x; 1.5346x over previous
import jax
import jax.numpy as jnp
from jax.experimental import pallas as pl

_BH = 32


def _copy_body(x_ref, o_ref):
    o_ref[...] = x_ref[...]


def kernel(x):
    n, c, h, w = x.shape
    y = jnp.transpose(x, (1, 2, 3, 0))
    out = pl.pallas_call(
        _copy_body,
        out_shape=jax.ShapeDtypeStruct((c, h, w, n), x.dtype),
        grid=(h // _BH,),
        in_specs=[pl.BlockSpec((c, _BH, w, n), lambda i: (0, i, 0, 0))],
        out_specs=pl.BlockSpec((c, _BH, w, n), lambda i: (0, i, 0, 0)),
    )(y)
    return jnp.transpose(out, (3, 0, 1, 2))
